# baseline (speedup 1.0000x reference)
import functools

import jax
import jax.numpy as jnp
from jax import lax
from jax.experimental import pallas as pl
from jax.experimental.pallas import tpu as pltpu
from jax.experimental.pallas import tpu_sc as plsc

N = 10000
E = 320000
H = 128
EPS = 1e-5

BLK = 10000
GRID = N // BLK

NCORE = 2
NSUB = 16
NW = NCORE * NSUB
EPW = E // NW
ECH = 80
KCH = EPW // ECH
NGRP = 5
GCH = KCH // NGRP
NSLOT = 3
RPT = 624
TAIL0 = NSUB * RPT
TAIL = N - TAIL0

_F32 = jnp.float32


def _gelu(v):
    return 0.5 * v * (1.0 + lax.erf(v * 0.7071067811865476))


def _fold(S, cs, W, b, g, beta):
    m = cs * (1.0 / N)
    mlin = jnp.dot(m, W, preferred_element_type=_F32)
    mean = mlin + b
    SW = jnp.dot(S, W, preferred_element_type=_F32) * (1.0 / N)
    ex2 = jnp.sum(W * SW, axis=0, keepdims=True)
    var = ex2 - mlin * mlin
    rstd = g * lax.rsqrt(var + EPS)
    return W * rstd, (b - mean) * rstd + beta


def _gram_acc(S_acc, cs_acc, u):
    S_acc[...] += lax.dot_general(u, u, (((0,), (0,)), ((), ())),
                                  preferred_element_type=_F32)
    cs_acc[...] += jnp.sum(u, axis=0, keepdims=True)


def _init_acc(*refs):
    for r in refs:
        r[...] = jnp.zeros(r.shape, r.dtype)


def _last():
    return pl.program_id(0) == pl.num_programs(0) - 1


def _k1_body(x_ref, W_ref, b_ref, g_ref, be_ref, Wf_ref, sh_ref, S_acc, cs_acc):
    @pl.when(pl.program_id(0) == 0)
    def _():
        _init_acc(S_acc, cs_acc)
    _gram_acc(S_acc, cs_acc, x_ref[...])

    @pl.when(_last())
    def _():
        Wf, sh = _fold(S_acc[...], cs_acc[...], W_ref[...], b_ref[...],
                       g_ref[...], be_ref[...])
        Wf_ref[...] = Wf
        sh_ref[...] = sh


def _k2_body(x_ref, Wf_ref, sh_ref, h_ref):
    h_ref[...] = jnp.dot(x_ref[...], Wf_ref[...],
                         preferred_element_type=_F32) + sh_ref[...]


def _sc_body(h_hbm, eidx_hbm, zero_hbm, out_hbm,
             src_v, dst_v, rows_v, acc_sh, gsem):
    c = lax.axis_index("c")
    s = lax.axis_index("s")
    w = c * NSUB + s
    r0 = s * RPT
    @pl.when(c == 0)
    def _():
        pltpu.sync_copy(h_hbm.at[pl.ds(r0, RPT)], acc_sh.at[pl.ds(r0, RPT)])

        @pl.when(s == NSUB - 1)
        def _():
            pltpu.sync_copy(h_hbm.at[pl.ds(TAIL0, TAIL)],
                            acc_sh.at[pl.ds(TAIL0, TAIL)])

    @pl.when(c != 0)
    def _():
        pltpu.sync_copy(zero_hbm.at[pl.ds(0, RPT)], acc_sh.at[pl.ds(r0, RPT)])

        @pl.when(s == NSUB - 1)
        def _():
            pltpu.sync_copy(zero_hbm.at[pl.ds(0, TAIL)],
                            acc_sh.at[pl.ds(TAIL0, TAIL)])
    plsc.subcore_barrier()

    def slot_view(slot):
        return rows_v.at[pl.ds(slot * ECH, ECH)]

    def gather(j, slot):
        pltpu.async_copy(h_hbm.at[src_v.at[j]], slot_view(slot), gsem.at[slot])

    def gwait(slot):
        pltpu.make_async_copy(h_hbm.at[src_v.at[0]], slot_view(slot),
                              gsem.at[slot]).wait()

    def group(g, carry):
        pltpu.sync_copy(eidx_hbm.at[0, w, g], src_v)
        pltpu.sync_copy(eidx_hbm.at[1, w, g], dst_v)
        gather(0, 0)
        gather(1, 1)

        def step(j, carry2):
            @pl.when(j + 2 < GCH)
            def _():
                gather(j + 2, (j + 2) % NSLOT)
            gwait(j % NSLOT)
            pltpu.sync_copy(slot_view(j % NSLOT), acc_sh.at[dst_v.at[j]],
                            add=True)
            return carry2

        lax.fori_loop(0, GCH, step, 0)
        return carry

    lax.fori_loop(0, NGRP, group, 0)
    plsc.subcore_barrier()
    pltpu.sync_copy(acc_sh.at[pl.ds(r0, RPT)], out_hbm.at[c, pl.ds(r0, RPT)])

    @pl.when(s == NSUB - 1)
    def _():
        pltpu.sync_copy(acc_sh.at[pl.ds(TAIL0, TAIL)],
                        out_hbm.at[c, pl.ds(TAIL0, TAIL)])


@functools.lru_cache(maxsize=1)
def _get_sc_scatter():
    return functools.partial(
        pl.kernel,
        out_type=jax.ShapeDtypeStruct((NCORE, N, H), _F32),
        mesh=plsc.VectorSubcoreMesh(core_axis_name="c", subcore_axis_name="s"),
        scratch_types=[
            pltpu.VMEM((GCH, ECH), jnp.int32),
            pltpu.VMEM((GCH, ECH), jnp.int32),
            pltpu.VMEM((NSLOT * ECH, H), _F32),
            pltpu.VMEM_SHARED((N, H), _F32),
            pltpu.SemaphoreType.DMA((NSLOT,)),
        ],
    )(_sc_body)


def _scatter_partials(h, eidx, zero):
    return _get_sc_scatter()(h, eidx, zero)


def _k3_body(agg_ref, W_ref, b_ref, g_ref, be_ref,
             z0_ref, Wf_ref, sh_ref, S_acc, cs_acc):
    @pl.when(pl.program_id(0) == 0)
    def _():
        _init_acc(S_acc, cs_acc)
    z0 = agg_ref[0] + agg_ref[1]
    z0_ref[...] = z0.astype(jnp.bfloat16)
    _gram_acc(S_acc, cs_acc, z0)

    @pl.when(_last())
    def _():
        Wf, sh = _fold(S_acc[...], cs_acc[...], W_ref[...], b_ref[...],
                       g_ref[...], be_ref[...])
        Wf_ref[...] = Wf
        sh_ref[...] = sh


def _k4_body(z0_ref, Wf2_ref, sh2_ref, W2_ref, b2_ref,
             Wn_ref, bn_ref, gn_ref, ben_ref,
             s_ref, Wf_ref, sh_ref, S_acc, cs_acc):
    @pl.when(pl.program_id(0) == 0)
    def _():
        _init_acc(S_acc, cs_acc)
    z1 = jnp.dot(z0_ref[...].astype(_F32), Wf2_ref[...],
                 preferred_element_type=_F32) + sh2_ref[...]
    u = _gelu(z1)
    sb = jnp.dot(u, W2_ref[...], preferred_element_type=_F32) + b2_ref[...]
    s_ref[...] = sb.astype(jnp.bfloat16)
    _gram_acc(S_acc, cs_acc, sb)

    @pl.when(_last())
    def _():
        Wf, sh = _fold(S_acc[...], cs_acc[...], Wn_ref[...], bn_ref[...],
                       gn_ref[...], ben_ref[...])
        Wf_ref[...] = Wf
        sh_ref[...] = sh


def _k5_body(s_ref, Wf3_ref, sh3_ref, W2_ref, b2_ref, g_ref, be_ref,
             Y_ref, sc_ref, sh_ref, cs_acc, css_acc):
    @pl.when(pl.program_id(0) == 0)
    def _():
        _init_acc(cs_acc, css_acc)
    y1 = jnp.dot(s_ref[...].astype(_F32), Wf3_ref[...],
                 preferred_element_type=_F32) + sh3_ref[...]
    v = _gelu(y1)
    Y = jnp.dot(v, W2_ref[...], preferred_element_type=_F32) + b2_ref[...]
    Y_ref[...] = Y.astype(jnp.bfloat16)
    cs_acc[...] += jnp.sum(Y, axis=0, keepdims=True)
    css_acc[...] += jnp.sum(Y * Y, axis=0, keepdims=True)

    @pl.when(_last())
    def _():
        mean = cs_acc[...] * (1.0 / N)
        var = css_acc[...] * (1.0 / N) - mean * mean
        scale = g_ref[...] * lax.rsqrt(var + EPS)
        sc_ref[...] = scale
        sh_ref[...] = be_ref[...] - mean * scale


def _k6_body(Y_ref, sc4_ref, sh4_ref, s_ref, x_ref,
             Wn_ref, bn_ref, gn_ref, ben_ref,
             y_ref, Wf_ref, sh_ref, S_acc, cs_acc):
    @pl.when(pl.program_id(0) == 0)
    def _():
        _init_acc(S_acc, cs_acc)
    y = (Y_ref[...].astype(_F32) * sc4_ref[...] + sh4_ref[...]
         + s_ref[...].astype(_F32) + x_ref[...])
    y_ref[...] = y.astype(jnp.bfloat16)
    _gram_acc(S_acc, cs_acc, y)

    @pl.when(_last())
    def _():
        Wf, sh = _fold(S_acc[...], cs_acc[...], Wn_ref[...], bn_ref[...],
                       gn_ref[...], ben_ref[...])
        Wf_ref[...] = Wf
        sh_ref[...] = sh


def _k7_body(y_ref, Wf5_ref, sh5_ref, W2_ref, b2_ref, g_ref, be_ref,
             Y_ref, sc_ref, sh_ref, cs_acc, css_acc):
    @pl.when(pl.program_id(0) == 0)
    def _():
        _init_acc(cs_acc, css_acc)
    t1 = jnp.dot(y_ref[...].astype(_F32), Wf5_ref[...],
                 preferred_element_type=_F32) + sh5_ref[...]
    w = _gelu(t1)
    Y = jnp.dot(w, W2_ref[...], preferred_element_type=_F32) + b2_ref[...]
    Y_ref[...] = Y.astype(jnp.bfloat16)
    cs_acc[...] += jnp.sum(Y, axis=0, keepdims=True)
    css_acc[...] += jnp.sum(Y * Y, axis=0, keepdims=True)

    @pl.when(_last())
    def _():
        mean = cs_acc[...] * (1.0 / N)
        var = css_acc[...] * (1.0 / N) - mean * mean
        scale = g_ref[...] * lax.rsqrt(var + EPS)
        sc_ref[...] = scale
        sh_ref[...] = be_ref[...] - mean * scale


def _k8_body(Y_ref, sc_ref, sh_ref, y_ref, o_ref):
    o_ref[...] = (Y_ref[...].astype(_F32) * sc_ref[...] + sh_ref[...]
                  + y_ref[...].astype(_F32))


def _blk(n_rows, cols):
    return pl.BlockSpec((n_rows, cols), lambda i: (i, 0))


def _full(r, c):
    return pl.BlockSpec((r, c), lambda i: (0, 0))


def _sds(shape, dtype=_F32):
    return jax.ShapeDtypeStruct(shape, dtype)


def _call(body, in_specs, out_specs, out_shapes, scratch):
    return pl.pallas_call(
        body,
        grid=(GRID,),
        in_specs=in_specs,
        out_specs=out_specs,
        out_shape=out_shapes,
        scratch_shapes=scratch,
    )


def kernel(x, edge_index, W_ne, b_ne, g_a, b_a, W_g1, b_g1, g_g, b_g,
           W_g2, b_g2, W_f1, b_f1, g_f1, be_f1, W_f2, b_f2, g_f2, be_f2,
           W_n1, b_n1, g_n1, be_n1, W_n2, b_n2, g_n2, be_n2):
    r = lambda v: v.reshape(1, -1)
    H2, H4 = 2 * H, 4 * H

    k1 = _call(_k1_body,
               [_blk(BLK, H), _full(H, H)] + [_full(1, H)] * 3,
               [_full(H, H), _full(1, H)],
               [_sds((H, H)), _sds((1, H))],
               [pltpu.VMEM((H, H), _F32), pltpu.VMEM((1, H), _F32)])
    Wf1, sh1 = k1(x, W_ne, r(b_ne), r(g_a), r(b_a))

    k2 = _call(_k2_body,
               [_blk(BLK, H), _full(H, H), _full(1, H)],
               [_blk(BLK, H)], [_sds((N, H))], [])
    h, = k2(x, Wf1, sh1)

    eidx = edge_index.reshape(2, NW, NGRP, GCH, ECH)
    aggp = _scatter_partials(h, eidx, jnp.zeros((RPT + TAIL, H), _F32))

    k3 = _call(_k3_body,
               [pl.BlockSpec((NCORE, BLK, H), lambda i: (0, i, 0)),
                _full(H, H)] + [_full(1, H)] * 3,
               [_blk(BLK, H), _full(H, H), _full(1, H)],
               [_sds((N, H), jnp.bfloat16), _sds((H, H)), _sds((1, H))],
               [pltpu.VMEM((H, H), _F32), pltpu.VMEM((1, H), _F32)])
    z0, Wf2, sh2 = k3(aggp, W_g1, r(b_g1), r(g_g), r(b_g))

    k4 = _call(_k4_body,
               [_blk(BLK, H), _full(H, H), _full(1, H), _full(H, H), _full(1, H),
                _full(H, H2), _full(1, H2), _full(1, H2), _full(1, H2)],
               [_blk(BLK, H), _full(H, H2), _full(1, H2)],
               [_sds((N, H), jnp.bfloat16), _sds((H, H2)), _sds((1, H2))],
               [pltpu.VMEM((H, H), _F32), pltpu.VMEM((1, H), _F32)])
    s, Wf3, sh3 = k4(z0, Wf2, sh2, W_g2, r(b_g2), W_f1, r(b_f1), r(g_f1), r(be_f1))

    k5 = _call(_k5_body,
               [_blk(BLK, H), _full(H, H2), _full(1, H2), _full(H2, H), _full(1, H),
                _full(1, H), _full(1, H)],
               [_blk(BLK, H), _full(1, H), _full(1, H)],
               [_sds((N, H), jnp.bfloat16), _sds((1, H)), _sds((1, H))],
               [pltpu.VMEM((1, H), _F32), pltpu.VMEM((1, H), _F32)])
    Y4, sc4, sh4 = k5(s, Wf3, sh3, W_f2, r(b_f2), r(g_f2), r(be_f2))

    k6 = _call(_k6_body,
               [_blk(BLK, H), _full(1, H), _full(1, H), _blk(BLK, H), _blk(BLK, H),
                _full(H, H4), _full(1, H4), _full(1, H4), _full(1, H4)],
               [_blk(BLK, H), _full(H, H4), _full(1, H4)],
               [_sds((N, H), jnp.bfloat16), _sds((H, H4)), _sds((1, H4))],
               [pltpu.VMEM((H, H), _F32), pltpu.VMEM((1, H), _F32)])
    y, Wf5, sh5 = k6(Y4, sc4, sh4, s, x, W_n1, r(b_n1), r(g_n1), r(be_n1))

    k7 = _call(_k7_body,
               [_blk(BLK, H), _full(H, H4), _full(1, H4), _full(H4, H), _full(1, H),
                _full(1, H), _full(1, H)],
               [_blk(BLK, H), _full(1, H), _full(1, H)],
               [_sds((N, H), jnp.bfloat16), _sds((1, H)), _sds((1, H))],
               [pltpu.VMEM((1, H), _F32), pltpu.VMEM((1, H), _F32)])
    Y6, sc6, sh6 = k7(y, Wf5, sh5, W_n2, r(b_n2), r(g_n2), r(be_n2))

    k8 = _call(_k8_body,
               [_blk(BLK, H), _full(1, H), _full(1, H), _blk(BLK, H)],
               [_blk(BLK, H)], [_sds((N, H))], [])
    out, = k8(Y6, sc6, sh6, y)
    return out


# speedup vs baseline: 1.0162x; 1.0162x over previous
import functools

import jax
import jax.numpy as jnp
from jax import lax
from jax.experimental import pallas as pl
from jax.experimental.pallas import tpu as pltpu
from jax.experimental.pallas import tpu_sc as plsc

N = 10000
E = 320000
H = 128
EPS = 1e-5

BLK = 5000
GRID = N // BLK

NCORE = 2
NSUB = 16
NW = NCORE * NSUB
EPW = E // NW
ECH = 80
KCH = EPW // ECH
NGRP = 5
GCH = KCH // NGRP
NSLOT = 3
RPT = 624
TAIL0 = NSUB * RPT
TAIL = N - TAIL0

_F32 = jnp.float32


def _gelu(v):
    return 0.5 * v * (1.0 + lax.erf(v * 0.7071067811865476))


def _fold(S, cs, W, b, g, beta):
    m = cs * (1.0 / N)
    mlin = jnp.dot(m, W, preferred_element_type=_F32)
    mean = mlin + b
    SW = jnp.dot(S, W, preferred_element_type=_F32) * (1.0 / N)
    ex2 = jnp.sum(W * SW, axis=0, keepdims=True)
    var = ex2 - mlin * mlin
    rstd = g * lax.rsqrt(var + EPS)
    return W * rstd, (b - mean) * rstd + beta


def _gram_acc(S_acc, cs_acc, u):
    S_acc[...] += lax.dot_general(u, u, (((0,), (0,)), ((), ())),
                                  preferred_element_type=_F32)
    cs_acc[...] += jnp.sum(u, axis=0, keepdims=True)


def _init_acc(*refs):
    for r in refs:
        r[...] = jnp.zeros(r.shape, r.dtype)


def _last():
    return pl.program_id(0) == pl.num_programs(0) - 1


def _k1_body(x_ref, W_ref, b_ref, g_ref, be_ref, Wf_ref, sh_ref, S_acc, cs_acc):
    @pl.when(pl.program_id(0) == 0)
    def _():
        _init_acc(S_acc, cs_acc)
    _gram_acc(S_acc, cs_acc, x_ref[...])

    @pl.when(_last())
    def _():
        Wf, sh = _fold(S_acc[...], cs_acc[...], W_ref[...], b_ref[...],
                       g_ref[...], be_ref[...])
        Wf_ref[...] = Wf
        sh_ref[...] = sh


def _k2_body(x_ref, Wf_ref, sh_ref, h_ref):
    h_ref[...] = jnp.dot(x_ref[...], Wf_ref[...],
                         preferred_element_type=_F32) + sh_ref[...]


def _sc_body(h_hbm, eidx_hbm, zero_hbm, out_hbm,
             src_v, dst_v, rows_v, acc_sh, gsem):
    c = lax.axis_index("c")
    s = lax.axis_index("s")
    w = c * NSUB + s
    r0 = s * RPT
    @pl.when(c == 0)
    def _():
        pltpu.sync_copy(h_hbm.at[pl.ds(r0, RPT)], acc_sh.at[pl.ds(r0, RPT)])

        @pl.when(s == NSUB - 1)
        def _():
            pltpu.sync_copy(h_hbm.at[pl.ds(TAIL0, TAIL)],
                            acc_sh.at[pl.ds(TAIL0, TAIL)])

    @pl.when(c != 0)
    def _():
        pltpu.sync_copy(zero_hbm.at[pl.ds(0, RPT)], acc_sh.at[pl.ds(r0, RPT)])

        @pl.when(s == NSUB - 1)
        def _():
            pltpu.sync_copy(zero_hbm.at[pl.ds(0, TAIL)],
                            acc_sh.at[pl.ds(TAIL0, TAIL)])
    plsc.subcore_barrier()

    def slot_view(slot):
        return rows_v.at[pl.ds(slot * ECH, ECH)]

    def gather(j, slot):
        pltpu.async_copy(h_hbm.at[src_v.at[j]], slot_view(slot), gsem.at[slot])

    def gwait(slot):
        pltpu.make_async_copy(h_hbm.at[src_v.at[0]], slot_view(slot),
                              gsem.at[slot]).wait()

    def group(g, carry):
        pltpu.sync_copy(eidx_hbm.at[0, w, g], src_v)
        pltpu.sync_copy(eidx_hbm.at[1, w, g], dst_v)
        gather(0, 0)
        gather(1, 1)

        def step(j, carry2):
            @pl.when(j + 2 < GCH)
            def _():
                gather(j + 2, (j + 2) % NSLOT)
            gwait(j % NSLOT)
            pltpu.sync_copy(slot_view(j % NSLOT), acc_sh.at[dst_v.at[j]],
                            add=True)
            return carry2

        lax.fori_loop(0, GCH, step, 0)
        return carry

    lax.fori_loop(0, NGRP, group, 0)
    plsc.subcore_barrier()
    pltpu.sync_copy(acc_sh.at[pl.ds(r0, RPT)], out_hbm.at[c, pl.ds(r0, RPT)])

    @pl.when(s == NSUB - 1)
    def _():
        pltpu.sync_copy(acc_sh.at[pl.ds(TAIL0, TAIL)],
                        out_hbm.at[c, pl.ds(TAIL0, TAIL)])


@functools.lru_cache(maxsize=1)
def _get_sc_scatter():
    return functools.partial(
        pl.kernel,
        out_type=jax.ShapeDtypeStruct((NCORE, N, H), _F32),
        mesh=plsc.VectorSubcoreMesh(core_axis_name="c", subcore_axis_name="s"),
        scratch_types=[
            pltpu.VMEM((GCH, ECH), jnp.int32),
            pltpu.VMEM((GCH, ECH), jnp.int32),
            pltpu.VMEM((NSLOT * ECH, H), _F32),
            pltpu.VMEM_SHARED((N, H), _F32),
            pltpu.SemaphoreType.DMA((NSLOT,)),
        ],
    )(_sc_body)


def _scatter_partials(h, eidx, zero):
    return _get_sc_scatter()(h, eidx, zero)


def _k3_body(agg_ref, W_ref, b_ref, g_ref, be_ref,
             z0_ref, Wf_ref, sh_ref, S_acc, cs_acc):
    @pl.when(pl.program_id(0) == 0)
    def _():
        _init_acc(S_acc, cs_acc)
    z0 = agg_ref[0] + agg_ref[1]
    z0_ref[...] = z0.astype(jnp.bfloat16)
    _gram_acc(S_acc, cs_acc, z0)

    @pl.when(_last())
    def _():
        Wf, sh = _fold(S_acc[...], cs_acc[...], W_ref[...], b_ref[...],
                       g_ref[...], be_ref[...])
        Wf_ref[...] = Wf
        sh_ref[...] = sh


def _k4_body(z0_ref, Wf2_ref, sh2_ref, W2_ref, b2_ref,
             Wn_ref, bn_ref, gn_ref, ben_ref,
             s_ref, Wf_ref, sh_ref, S_acc, cs_acc):
    @pl.when(pl.program_id(0) == 0)
    def _():
        _init_acc(S_acc, cs_acc)
    z1 = jnp.dot(z0_ref[...].astype(_F32), Wf2_ref[...],
                 preferred_element_type=_F32) + sh2_ref[...]
    u = _gelu(z1)
    sb = jnp.dot(u, W2_ref[...], preferred_element_type=_F32) + b2_ref[...]
    s_ref[...] = sb.astype(jnp.bfloat16)
    _gram_acc(S_acc, cs_acc, sb)

    @pl.when(_last())
    def _():
        Wf, sh = _fold(S_acc[...], cs_acc[...], Wn_ref[...], bn_ref[...],
                       gn_ref[...], ben_ref[...])
        Wf_ref[...] = Wf
        sh_ref[...] = sh


def _k5_body(s_ref, Wf3_ref, sh3_ref, W2_ref, b2_ref, g_ref, be_ref,
             Y_ref, sc_ref, sh_ref, cs_acc, css_acc):
    @pl.when(pl.program_id(0) == 0)
    def _():
        _init_acc(cs_acc, css_acc)
    y1 = jnp.dot(s_ref[...].astype(_F32), Wf3_ref[...],
                 preferred_element_type=_F32) + sh3_ref[...]
    v = _gelu(y1)
    Y = jnp.dot(v, W2_ref[...], preferred_element_type=_F32) + b2_ref[...]
    Y_ref[...] = Y.astype(jnp.bfloat16)
    cs_acc[...] += jnp.sum(Y, axis=0, keepdims=True)
    css_acc[...] += jnp.sum(Y * Y, axis=0, keepdims=True)

    @pl.when(_last())
    def _():
        mean = cs_acc[...] * (1.0 / N)
        var = css_acc[...] * (1.0 / N) - mean * mean
        scale = g_ref[...] * lax.rsqrt(var + EPS)
        sc_ref[...] = scale
        sh_ref[...] = be_ref[...] - mean * scale


def _k6_body(Y_ref, sc4_ref, sh4_ref, s_ref, x_ref,
             Wn_ref, bn_ref, gn_ref, ben_ref,
             y_ref, Wf_ref, sh_ref, S_acc, cs_acc):
    @pl.when(pl.program_id(0) == 0)
    def _():
        _init_acc(S_acc, cs_acc)
    y = (Y_ref[...].astype(_F32) * sc4_ref[...] + sh4_ref[...]
         + s_ref[...].astype(_F32) + x_ref[...])
    y_ref[...] = y.astype(jnp.bfloat16)
    _gram_acc(S_acc, cs_acc, y)

    @pl.when(_last())
    def _():
        Wf, sh = _fold(S_acc[...], cs_acc[...], Wn_ref[...], bn_ref[...],
                       gn_ref[...], ben_ref[...])
        Wf_ref[...] = Wf
        sh_ref[...] = sh


def _k7_body(y_ref, Wf5_ref, sh5_ref, W2_ref, b2_ref, g_ref, be_ref,
             Y_ref, sc_ref, sh_ref, cs_acc, css_acc):
    @pl.when(pl.program_id(0) == 0)
    def _():
        _init_acc(cs_acc, css_acc)
    t1 = jnp.dot(y_ref[...].astype(_F32), Wf5_ref[...],
                 preferred_element_type=_F32) + sh5_ref[...]
    w = _gelu(t1)
    Y = jnp.dot(w, W2_ref[...], preferred_element_type=_F32) + b2_ref[...]
    Y_ref[...] = Y.astype(jnp.bfloat16)
    cs_acc[...] += jnp.sum(Y, axis=0, keepdims=True)
    css_acc[...] += jnp.sum(Y * Y, axis=0, keepdims=True)

    @pl.when(_last())
    def _():
        mean = cs_acc[...] * (1.0 / N)
        var = css_acc[...] * (1.0 / N) - mean * mean
        scale = g_ref[...] * lax.rsqrt(var + EPS)
        sc_ref[...] = scale
        sh_ref[...] = be_ref[...] - mean * scale


def _k8_body(Y_ref, sc_ref, sh_ref, y_ref, o_ref):
    o_ref[...] = (Y_ref[...].astype(_F32) * sc_ref[...] + sh_ref[...]
                  + y_ref[...].astype(_F32))


def _blk(n_rows, cols):
    return pl.BlockSpec((n_rows, cols), lambda i: (i, 0))


def _full(r, c):
    return pl.BlockSpec((r, c), lambda i: (0, 0))


def _sds(shape, dtype=_F32):
    return jax.ShapeDtypeStruct(shape, dtype)


def _call(body, in_specs, out_specs, out_shapes, scratch):
    return pl.pallas_call(
        body,
        grid=(GRID,),
        in_specs=in_specs,
        out_specs=out_specs,
        out_shape=out_shapes,
        scratch_shapes=scratch,
    )


def kernel(x, edge_index, W_ne, b_ne, g_a, b_a, W_g1, b_g1, g_g, b_g,
           W_g2, b_g2, W_f1, b_f1, g_f1, be_f1, W_f2, b_f2, g_f2, be_f2,
           W_n1, b_n1, g_n1, be_n1, W_n2, b_n2, g_n2, be_n2):
    r = lambda v: v.reshape(1, -1)
    H2, H4 = 2 * H, 4 * H

    k1 = _call(_k1_body,
               [_blk(BLK, H), _full(H, H)] + [_full(1, H)] * 3,
               [_full(H, H), _full(1, H)],
               [_sds((H, H)), _sds((1, H))],
               [pltpu.VMEM((H, H), _F32), pltpu.VMEM((1, H), _F32)])
    Wf1, sh1 = k1(x, W_ne, r(b_ne), r(g_a), r(b_a))

    k2 = _call(_k2_body,
               [_blk(BLK, H), _full(H, H), _full(1, H)],
               [_blk(BLK, H)], [_sds((N, H))], [])
    h, = k2(x, Wf1, sh1)

    eidx = edge_index.reshape(2, NW, NGRP, GCH, ECH)
    aggp = _scatter_partials(h, eidx, jnp.zeros((RPT + TAIL, H), _F32))

    k3 = _call(_k3_body,
               [pl.BlockSpec((NCORE, BLK, H), lambda i: (0, i, 0)),
                _full(H, H)] + [_full(1, H)] * 3,
               [_blk(BLK, H), _full(H, H), _full(1, H)],
               [_sds((N, H), jnp.bfloat16), _sds((H, H)), _sds((1, H))],
               [pltpu.VMEM((H, H), _F32), pltpu.VMEM((1, H), _F32)])
    z0, Wf2, sh2 = k3(aggp, W_g1, r(b_g1), r(g_g), r(b_g))

    k4 = _call(_k4_body,
               [_blk(BLK, H), _full(H, H), _full(1, H), _full(H, H), _full(1, H),
                _full(H, H2), _full(1, H2), _full(1, H2), _full(1, H2)],
               [_blk(BLK, H), _full(H, H2), _full(1, H2)],
               [_sds((N, H), jnp.bfloat16), _sds((H, H2)), _sds((1, H2))],
               [pltpu.VMEM((H, H), _F32), pltpu.VMEM((1, H), _F32)])
    s, Wf3, sh3 = k4(z0, Wf2, sh2, W_g2, r(b_g2), W_f1, r(b_f1), r(g_f1), r(be_f1))

    k5 = _call(_k5_body,
               [_blk(BLK, H), _full(H, H2), _full(1, H2), _full(H2, H), _full(1, H),
                _full(1, H), _full(1, H)],
               [_blk(BLK, H), _full(1, H), _full(1, H)],
               [_sds((N, H), jnp.bfloat16), _sds((1, H)), _sds((1, H))],
               [pltpu.VMEM((1, H), _F32), pltpu.VMEM((1, H), _F32)])
    Y4, sc4, sh4 = k5(s, Wf3, sh3, W_f2, r(b_f2), r(g_f2), r(be_f2))

    k6 = _call(_k6_body,
               [_blk(BLK, H), _full(1, H), _full(1, H), _blk(BLK, H), _blk(BLK, H),
                _full(H, H4), _full(1, H4), _full(1, H4), _full(1, H4)],
               [_blk(BLK, H), _full(H, H4), _full(1, H4)],
               [_sds((N, H), jnp.bfloat16), _sds((H, H4)), _sds((1, H4))],
               [pltpu.VMEM((H, H), _F32), pltpu.VMEM((1, H), _F32)])
    y, Wf5, sh5 = k6(Y4, sc4, sh4, s, x, W_n1, r(b_n1), r(g_n1), r(be_n1))

    k7 = _call(_k7_body,
               [_blk(BLK, H), _full(H, H4), _full(1, H4), _full(H4, H), _full(1, H),
                _full(1, H), _full(1, H)],
               [_blk(BLK, H), _full(1, H), _full(1, H)],
               [_sds((N, H), jnp.bfloat16), _sds((1, H)), _sds((1, H))],
               [pltpu.VMEM((1, H), _F32), pltpu.VMEM((1, H), _F32)])
    Y6, sc6, sh6 = k7(y, Wf5, sh5, W_n2, r(b_n2), r(g_n2), r(be_n2))

    k8 = _call(_k8_body,
               [_blk(BLK, H), _full(1, H), _full(1, H), _blk(BLK, H)],
               [_blk(BLK, H)], [_sds((N, H))], [])
    out, = k8(Y6, sc6, sh6, y)
    return out
